# baseline (speedup 1.0000x reference)
import functools

import jax
import jax.numpy as jnp
from jax import lax
from jax.experimental import pallas as pl
from jax.experimental.pallas import tpu as pltpu
from jax.experimental.pallas import tpu_sc as plsc

VOCAB = 100000
EMBED = 128
MAXLEN = 200
BATCH = 1024
SEQ = 200

_INFO = plsc.get_sparse_core_info()
_NC = _INFO.num_cores
_NS = _INFO.num_subcores
_NW = _NC * _NS
_ROWS = BATCH * SEQ
_RPW = _ROWS // _NW
_CHUNK = 80
_NCHUNK = _RPW // _CHUNK
_NB = 5
_LANES = 16
_VECS = EMBED // _LANES


def _embed_kernel(x_hbm, wt_hbm, pos_hbm, out_hbm,
                  idx_v, buf0, buf1, buf2, buf3, buf4, pos2_v,
                  gsem0, gsem1, gsem2, gsem3, gsem4,
                  ssem0, ssem1, ssem2, ssem3, ssem4):
    wid = lax.axis_index("s") * _NC + lax.axis_index("c")
    base = wid * _RPW
    bufs = (buf0, buf1, buf2, buf3, buf4)
    gsems = (gsem0, gsem1, gsem2, gsem3, gsem4)
    ssems = (ssem0, ssem1, ssem2, ssem3, ssem4)

    pltpu.sync_copy(x_hbm.at[pl.ds(base, _RPW)], idx_v)
    pltpu.sync_copy(pos_hbm, pos2_v.at[pl.ds(0, MAXLEN)])
    pltpu.sync_copy(pos_hbm, pos2_v.at[pl.ds(MAXLEN, MAXLEN)])

    def gather(c, b):
        return pltpu.make_async_copy(
            wt_hbm.at[idx_v.at[pl.ds(c * _CHUNK, _CHUNK)]], bufs[b], gsems[b])

    def store(c, b):
        return pltpu.make_async_copy(
            bufs[b], out_hbm.at[pl.ds(base + c * _CHUNK, _CHUNK)], ssems[b])

    def add_pos(c, b):
        p0 = lax.rem(c * _CHUNK, SEQ)
        buf = bufs[b]

        @plsc.parallel_loop(0, _CHUNK, step=1, unroll=8)
        def _(i):
            for k in range(_VECS):
                v = pos2_v[p0 + i, pl.ds(k * _LANES, _LANES)]
                plsc.addupdate(buf.at[i, pl.ds(k * _LANES, _LANES)], v)

    for c in range(_NB - 1):
        gather(c, c).start()

    def ring_body(g4, carry):
        g = g4 * _NB
        for b in range(_NB):
            c = g + b
            gather(c, b).wait()
            add_pos(c, b)
            store(c, b).start()

            @pl.when(c >= 1)
            def _():
                store(c - 1, (b + _NB - 1) % _NB).wait()

            @pl.when(c + _NB - 1 < _NCHUNK)
            def _():
                gather(c + _NB - 1, (b + _NB - 1) % _NB).start()
        return carry

    lax.fori_loop(0, _NCHUNK // _NB, ring_body, 0, unroll=False)

    store(_NCHUNK - 1, (_NCHUNK - 1) % _NB).wait()


@jax.jit
def _run(x_flat, word_table, pos_table):
    mesh = plsc.VectorSubcoreMesh(core_axis_name="c", subcore_axis_name="s")
    f = functools.partial(
        pl.kernel,
        mesh=mesh,
        out_type=jax.ShapeDtypeStruct((_ROWS, EMBED), jnp.float32),
        scratch_types=[
            pltpu.VMEM((_RPW,), jnp.int32),
            pltpu.VMEM((_CHUNK, EMBED), jnp.float32),
            pltpu.VMEM((_CHUNK, EMBED), jnp.float32),
            pltpu.VMEM((_CHUNK, EMBED), jnp.float32),
            pltpu.VMEM((_CHUNK, EMBED), jnp.float32),
            pltpu.VMEM((_CHUNK, EMBED), jnp.float32),
            pltpu.VMEM((2 * MAXLEN, EMBED), jnp.float32),
            pltpu.SemaphoreType.DMA,
            pltpu.SemaphoreType.DMA,
            pltpu.SemaphoreType.DMA,
            pltpu.SemaphoreType.DMA,
            pltpu.SemaphoreType.DMA,
            pltpu.SemaphoreType.DMA,
            pltpu.SemaphoreType.DMA,
            pltpu.SemaphoreType.DMA,
            pltpu.SemaphoreType.DMA,
            pltpu.SemaphoreType.DMA,
        ],
    )(_embed_kernel)
    return f(x_flat, word_table, pos_table)


def kernel(x, word_table, pos_table):
    x_flat = x.reshape(-1).astype(jnp.int32)
    out = _run(x_flat, word_table, pos_table)
    return out.reshape(BATCH, SEQ, EMBED)
